# baseline (speedup 1.0000x reference)
import functools

import jax
import jax.numpy as jnp
from jax import lax
from jax.experimental import pallas as pl
from jax.experimental.pallas import tpu as pltpu
from jax.experimental.pallas import tpu_sc as plsc

B_ = 4096
L_ = 200
H_ = 128
V_ = 32
N_ = B_ * L_

NC_ = 2
NS_ = 16
NW = NC_ * NS_
NPW = N_ // NW
C_ = 128
NB_ = 4
NCH = NPW // C_
TR_ = V_ + 8
LANES = 16


def _table_body(tok_ref, atom_ref, out_ref):
    out_ref[0:V_, :] = tok_ref[:, :]
    s = jnp.sum(atom_ref[:, :], axis=0, keepdims=True)
    out_ref[V_:TR_, :] = jnp.broadcast_to(s, (TR_ - V_, H_))


_build_table = pl.pallas_call(
    _table_body,
    out_shape=jax.ShapeDtypeStruct((TR_, H_), jnp.float32),
)


def _lookup_body(x_hbm, m_hbm, table_hbm, out_hbm, idx_v, m_v, rows_v, spm,
                 gsem0, gsem1, gsem2, gsem3, ssem0, ssem1, ssem2, ssem3):
    gsems = (gsem0, gsem1, gsem2, gsem3)
    ssems = (ssem0, ssem1, ssem2, ssem3)
    cid = lax.axis_index("c")
    sid = lax.axis_index("s")
    wid = sid * NC_ + cid
    base = wid * NPW

    pltpu.sync_copy(table_hbm, spm.at[pl.ds(sid * TR_, TR_)])
    pltpu.sync_copy(x_hbm.at[pl.ds(base, NPW)], idx_v)
    pltpu.sync_copy(m_hbm.at[pl.ds(base, NPW)], m_v)

    mask_idx = jnp.full((LANES,), V_, jnp.int32)
    off = sid * TR_

    @pl.loop(0, NPW // LANES)
    def _sel(i):
        sl = pl.ds(i * LANES, LANES)
        idx_v[sl] = jnp.where(m_v[sl] != 0, mask_idx, idx_v[sl]) + off

    def _gather(g, b):
        return pltpu.make_async_copy(
            spm.at[idx_v.at[pl.ds(g * C_, C_)]], rows_v.at[b], gsems[b])

    def _scatter(g, b):
        return pltpu.make_async_copy(
            rows_v.at[b], out_hbm.at[pl.ds(base + g * C_, C_)], ssems[b])

    @pl.loop(0, NCH // NB_)
    def _pipe(ki):
        for b in range(NB_):
            g = ki * NB_ + b

            @pl.when(ki > 0)
            def _():
                _scatter(g - NB_, b).wait()

            _gather(g, b).start()
            _gather(g, b).wait()
            _scatter(g, b).start()

    for b in range(NB_):
        _scatter(NCH - NB_ + b, b).wait()


_lookup = functools.partial(
    pl.kernel,
    mesh=plsc.VectorSubcoreMesh(core_axis_name="c", subcore_axis_name="s"),
    out_type=jax.ShapeDtypeStruct((N_, H_), jnp.float32),
    scratch_types=[
        pltpu.VMEM((NPW,), jnp.int32),
        pltpu.VMEM((NPW,), jnp.int32),
        pltpu.VMEM((NB_, C_, H_), jnp.float32),
        pltpu.VMEM_SHARED((NS_ * TR_, H_), jnp.float32),
    ] + [pltpu.SemaphoreType.DMA] * (2 * NB_),
)(_lookup_body)


def kernel(x, mask_aa, token_embed, atom_mask_embedding):
    xf = x.reshape(N_).astype(jnp.int32)
    mf = mask_aa.reshape(N_).astype(jnp.int32)
    table = _build_table(token_embed, atom_mask_embedding)
    out = _lookup(xf, mf, table)
    return out.reshape(B_, L_, H_)
